# baseline (device time: 236767 ns/iter reference)
import jax
import jax.numpy as jnp
from jax import lax
from jax.experimental import pallas as pl
from jax.experimental.pallas import tpu as pltpu

M, N = 16384, 2048
MH, NH = M // 2, N // 2

SIZES = [128, 256] + [608] * 12 + [256, 128, 64, 64]
assert sum(SIZES) == MH
C = len(SIZES)
OFFS = [sum(SIZES[:i]) for i in range(C)]
MAXCH = max(SIZES)

YGROUPS = [(0, 1), (1, 2), (2, 4), (4, 6), (6, 8), (8, 10), (10, 12),
           (12, 14), (14, 18)]
assert YGROUPS[0][0] == 0 and YGROUPS[-1][1] == C
assert all(a[1] == b[0] for a, b in zip(YGROUPS[:-1], YGROUPS[1:]))
G = len(YGROUPS)


def kernel(x):
    def body(x_ref, out_ref, send_ref, recvx_ref, recvy_ref,
             stage_p, stage_m, lp_sems, lm_sems, store_sems, store2_sems,
             sx_send, sx_recv, sy_send, sy_recv):
        mx = lax.axis_index("x")
        my = lax.axis_index("y")
        rows0 = my * MH
        prow0 = (1 - my) * MH
        mcol = mx * NH
        pcol = (1 - mx) * NH

        barrier_sem = pltpu.get_barrier_semaphore()
        for nbr in ((1 - mx, my), (mx, 1 - my)):
            pl.semaphore_signal(
                barrier_sem, inc=1,
                device_id=nbr, device_id_type=pl.DeviceIdType.MESH,
            )
        pl.semaphore_wait(barrier_sem, 2)

        def load(c, stage, sems, row0, col0):
            return pltpu.make_async_copy(
                x_ref.at[0, pl.ds(row0 + OFFS[c], SIZES[c]), pl.ds(col0, NH)],
                stage.at[c % 2, pl.ds(0, SIZES[c]), :], sems.at[c % 2])

        loads_p = [load(c, stage_p, lp_sems, rows0, pcol) for c in range(C)]
        rdmas_x = []
        loads_p[0].start()
        for c in range(C):
            off, sz = OFFS[c], SIZES[c]
            if c + 1 < C:
                loads_p[c + 1].start()
            loads_p[c].wait()
            send_ref[pl.ds(off, sz), :] = (
                stage_p[c % 2, pl.ds(0, sz), :].astype(jnp.bfloat16))
            r = pltpu.make_async_remote_copy(
                src_ref=send_ref.at[pl.ds(off, sz), :],
                dst_ref=recvx_ref.at[pl.ds(off, sz), :],
                send_sem=sx_send.at[c], recv_sem=sx_recv.at[c],
                device_id=(1 - mx, my), device_id_type=pl.DeviceIdType.MESH,
            )
            r.start()
            rdmas_x.append(r)

        loads_m = [load(c, stage_m, lm_sems, rows0, mcol) for c in range(C)]
        stores = []
        rdmas_y = []
        loads_m[0].start()
        for g, (cs, ce) in enumerate(YGROUPS):
            for c in range(cs, ce):
                off, sz = OFFS[c], SIZES[c]
                if c + 1 < C:
                    loads_m[c + 1].start()
                loads_m[c].wait()
                rdmas_x[c].wait_recv()
                rdmas_x[c].wait_send()
                send_ref[pl.ds(off, sz), :] = (
                    recvx_ref[pl.ds(off, sz), :]
                    + stage_m[c % 2, pl.ds(0, sz), :].astype(jnp.bfloat16))
            goff = OFFS[cs]
            gsz = sum(SIZES[cs:ce])
            ry = pltpu.make_async_remote_copy(
                src_ref=send_ref.at[pl.ds(goff, gsz), :],
                dst_ref=recvy_ref.at[pl.ds(goff, gsz), :],
                send_sem=sy_send.at[g], recv_sem=sy_recv.at[g],
                device_id=(mx, 1 - my), device_id_type=pl.DeviceIdType.MESH,
            )
            ry.start()
            rdmas_y.append(ry)
            st = pltpu.make_async_copy(
                send_ref.at[pl.ds(goff, gsz), :],
                out_ref.at[pl.ds(rows0 + goff, gsz), :],
                store_sems.at[g],
            )
            st.start()
            stores.append(st)

        stores2 = []
        for g, (cs, ce) in enumerate(YGROUPS):
            goff = OFFS[cs]
            gsz = sum(SIZES[cs:ce])
            rdmas_y[g].wait_recv()
            st2 = pltpu.make_async_copy(
                recvy_ref.at[pl.ds(goff, gsz), :],
                out_ref.at[pl.ds(prow0 + goff, gsz), :],
                store2_sems.at[g],
            )
            st2.start()
            stores2.append(st2)

        for g in range(G):
            rdmas_y[g].wait_send()
            stores[g].wait()
            stores2[g].wait()

    return pl.pallas_call(
        body,
        out_shape=jax.ShapeDtypeStruct((M, NH), jnp.bfloat16),
        in_specs=[pl.BlockSpec(memory_space=pl.ANY)],
        out_specs=pl.BlockSpec(memory_space=pl.ANY),
        scratch_shapes=[
            pltpu.VMEM((MH, NH), jnp.bfloat16),
            pltpu.VMEM((MH, NH), jnp.bfloat16),
            pltpu.VMEM((MH, NH), jnp.bfloat16),
            pltpu.VMEM((2, MAXCH, NH), jnp.float32),
            pltpu.VMEM((2, MAXCH, NH), jnp.float32),
            pltpu.SemaphoreType.DMA((2,)),
            pltpu.SemaphoreType.DMA((2,)),
            pltpu.SemaphoreType.DMA((G,)),
            pltpu.SemaphoreType.DMA((G,)),
            pltpu.SemaphoreType.DMA((C,)),
            pltpu.SemaphoreType.DMA((C,)),
            pltpu.SemaphoreType.DMA((G,)),
            pltpu.SemaphoreType.DMA((G,)),
        ],
        compiler_params=pltpu.CompilerParams(
            collective_id=0, vmem_limit_bytes=63 * 1024 * 1024),
    )(x)


# device time: 223687 ns/iter; 1.0585x vs baseline; 1.0585x over previous
import jax
import jax.numpy as jnp
from jax import lax
from jax.experimental import pallas as pl
from jax.experimental.pallas import tpu as pltpu

M, N = 16384, 2048
MH, NH = M // 2, N // 2
C = 16
CH = MH // C


def kernel(x):
    def body(x_ref, out_ref, send_ref, recvx_ref, stage_p, stage_m,
             lp_sems, lm_sems, store_sems, sx_send, sx_recv,
             sy_send, sy_recv):
        mx = lax.axis_index("x")
        my = lax.axis_index("y")
        rows0 = my * MH
        mcol = mx * NH
        pcol = (1 - mx) * NH

        barrier_sem = pltpu.get_barrier_semaphore()
        for nbr in ((1 - mx, my), (mx, 1 - my)):
            pl.semaphore_signal(
                barrier_sem, inc=1,
                device_id=nbr, device_id_type=pl.DeviceIdType.MESH,
            )
        pl.semaphore_wait(barrier_sem, 2)

        def load(c, stage, sems, col0):
            return pltpu.make_async_copy(
                x_ref.at[0, pl.ds(rows0 + c * CH, CH), pl.ds(col0, NH)],
                stage.at[c % 2], sems.at[c % 2])

        loads_p = [load(c, stage_p, lp_sems, pcol) for c in range(C)]
        rdmas_x = []
        loads_p[0].start()
        for c in range(C):
            if c + 1 < C:
                loads_p[c + 1].start()
            loads_p[c].wait()
            send_ref[pl.ds(c * CH, CH), :] = (
                stage_p[c % 2].astype(jnp.bfloat16))
            r = pltpu.make_async_remote_copy(
                src_ref=send_ref.at[pl.ds(c * CH, CH), :],
                dst_ref=recvx_ref.at[pl.ds(c * CH, CH), :],
                send_sem=sx_send.at[c], recv_sem=sx_recv.at[c],
                device_id=(1 - mx, my), device_id_type=pl.DeviceIdType.MESH,
            )
            r.start()
            rdmas_x.append(r)

        loads_m = [load(c, stage_m, lm_sems, mcol) for c in range(C)]
        stores = []
        rdmas_y = []
        loads_m[0].start()
        for c in range(C):
            if c + 1 < C:
                loads_m[c + 1].start()
            loads_m[c].wait()
            rdmas_x[c].wait_recv()
            recvx_ref[pl.ds(c * CH, CH), :] = (
                recvx_ref[pl.ds(c * CH, CH), :]
                + stage_m[c % 2].astype(jnp.bfloat16))
            st = pltpu.make_async_copy(
                recvx_ref.at[pl.ds(c * CH, CH), :],
                out_ref.at[pl.ds(rows0 + c * CH, CH), :],
                store_sems.at[c],
            )
            st.start()
            stores.append(st)
            ry = pltpu.make_async_remote_copy(
                src_ref=recvx_ref.at[pl.ds(c * CH, CH), :],
                dst_ref=out_ref.at[pl.ds(rows0 + c * CH, CH), :],
                send_sem=sy_send.at[c], recv_sem=sy_recv.at[c],
                device_id=(mx, 1 - my), device_id_type=pl.DeviceIdType.MESH,
            )
            ry.start()
            rdmas_y.append(ry)

        for c in range(C):
            rdmas_x[c].wait_send()
            stores[c].wait()
            rdmas_y[c].wait()

    return pl.pallas_call(
        body,
        out_shape=jax.ShapeDtypeStruct((M, NH), jnp.bfloat16),
        in_specs=[pl.BlockSpec(memory_space=pl.ANY)],
        out_specs=pl.BlockSpec(memory_space=pl.ANY),
        scratch_shapes=[
            pltpu.VMEM((MH, NH), jnp.bfloat16),
            pltpu.VMEM((MH, NH), jnp.bfloat16),
            pltpu.VMEM((2, CH, NH), jnp.float32),
            pltpu.VMEM((2, CH, NH), jnp.float32),
            pltpu.SemaphoreType.DMA((2,)),
            pltpu.SemaphoreType.DMA((2,)),
            pltpu.SemaphoreType.DMA((C,)),
            pltpu.SemaphoreType.DMA((C,)),
            pltpu.SemaphoreType.DMA((C,)),
            pltpu.SemaphoreType.DMA((C,)),
            pltpu.SemaphoreType.DMA((C,)),
        ],
        compiler_params=pltpu.CompilerParams(
            collective_id=0, vmem_limit_bytes=63 * 1024 * 1024),
    )(x)
